# baseline (speedup 1.0000x reference)
import functools
from math import log

import jax
import jax.numpy as jnp
from jax import lax
from jax.experimental import pallas as pl
from jax.experimental.pallas import tpu as pltpu
from jax.experimental.pallas import tpu_sc as plsc

ALPHA = 0.1
BETA = log(0.5 / (2 + 1) + 1)

NC = 2
NS = 16
L = 16

K = 80


_DNUMS = lax.GatherDimensionNumbers(
    offset_dims=(), collapsed_slice_dims=(0,), start_index_map=(0,))


def _sc_segment_sum(x2, src, dst, w, N):
    E = src.shape[0]
    ept = E // NS
    nch = ept // K
    assert E % NS == 0 and ept % K == 0 and K % 8 == 0
    assert nch % 2 == 1 and nch >= 7
    ncha = (nch // 2) if (nch // 2) % 2 == 0 else (nch // 2 + 1)
    nchb = nch - ncha
    dst_buf = max(ncha, nchb) * K
    num_groups = N // K
    assert N % K == 0
    mesh = plsc.VectorSubcoreMesh(core_axis_name="c", subcore_axis_name="s")

    @functools.partial(
        pl.kernel,
        mesh=mesh,
        out_type=jax.ShapeDtypeStruct((N, NC * 128), jnp.float32),
        scratch_types=[
            pltpu.VMEM((dst_buf,), jnp.int32),
            pltpu.VMEM((K,), jnp.int32),
            pltpu.VMEM((K,), jnp.int32),
            pltpu.VMEM((K,), jnp.float32),
            pltpu.VMEM((K,), jnp.float32),
            pltpu.VMEM((K,), jnp.int32),
            pltpu.VMEM((K,), jnp.int32),
            pltpu.VMEM((K,), jnp.int32),
            pltpu.VMEM((K,), jnp.int32),
            pltpu.VMEM((K, 128), jnp.float32),
            pltpu.VMEM((K, 128), jnp.float32),
            pltpu.VMEM((K, 128), jnp.float32),
            pltpu.VMEM((K, 128), jnp.float32),
            pltpu.VMEM_SHARED((N, 128), jnp.float32),
            pltpu.SemaphoreType.DMA,
            pltpu.SemaphoreType.DMA,
            pltpu.SemaphoreType.DMA,
            pltpu.SemaphoreType.DMA,
            pltpu.SemaphoreType.DMA,
            pltpu.SemaphoreType.DMA,
        ],
    )
    def seg_sum(x2_hbm, src_hbm, dst_hbm, w_hbm, out_hbm,
                didx_st, sidx0, sidx1, wch0, wch1, gidx0, gidx1,
                didx0, didx1, graw0, graw1, rows0, rows1, acc_sh,
                gsem0, gsem1, csem0, csem1, ssem0, ssem1):
        c = lax.axis_index("c")
        s = lax.axis_index("s")
        sidx = (sidx0, sidx1)
        wch = (wch0, wch1)
        gidx = (gidx0, gidx1)
        didx = (didx0, didx1)
        graw = (graw0, graw1)
        rows = (rows0, rows1)
        gsem = (gsem0, gsem1)
        csem = (csem0, csem1)
        ssem = (ssem0, ssem1)
        ebase = s * ept

        pltpu.sync_copy(dst_hbm.at[pl.ds(ebase, ncha * K)],
                        didx_st.at[pl.ds(0, ncha * K)])

        zero16 = jnp.zeros((L,), jnp.float32)

        @pl.loop(0, K)
        def _zero_rows(r):
            for j in range(128 // L):
                rows0[r, pl.ds(j * L, L)] = zero16

        @pl.loop(s, num_groups, step=NS)
        def _zero_acc(g):
            pltpu.sync_copy(rows0, acc_sh.at[pl.ds(g * K, K)])

        plsc.subcore_barrier()

        def issue_src(j, b):
            pltpu.async_copy(src_hbm.at[pl.ds(ebase + j * K, K)],
                             sidx[b], csem[b])
            pltpu.async_copy(w_hbm.at[pl.ds(ebase + j * K, K)],
                             wch[b], csem[b])

        def wait_src(j, b):
            pltpu.make_async_copy(src_hbm.at[pl.ds(ebase + j * K, K)],
                                  sidx[b], csem[b]).wait()
            pltpu.make_async_copy(w_hbm.at[pl.ds(ebase + j * K, K)],
                                  wch[b], csem[b]).wait()

        def issue_gather(j, b):
            for i in range(K // L):
                sl = pl.ds(i * L, L)
                gidx[b][sl] = sidx[b][sl] * 2 + c
            pltpu.async_copy(x2_hbm.at[gidx[b]], graw[b], gsem[b])

        def wait_gather(b):
            pltpu.make_async_copy(x2_hbm.at[gidx[b]], graw[b], gsem[b]).wait()

        def wait_scatter(b):
            pltpu.make_async_copy(rows[b], acc_sh.at[didx[b]], ssem[b]).wait()

        def chunk(j, b, pbase, first=False, issue_src2=True, issue_next=True):
            b2 = 1 - b
            if issue_next:
                wait_src(j + 1, b2)
                issue_gather(j + 1, b2)
            wait_gather(b)
            if not first:
                wait_scatter(b)

            @pl.loop(0, K, step=L)
            def _wblk(bb):
                wv = wch[b][pl.ds(bb, L)]

                @pl.loop(0, L, step=4)
                def _edges(e0):
                    for de in range(4):
                        e = e0 + de
                        wvec = lax.gather(
                            wv, jnp.full((L, 1), e, jnp.int32), _DNUMS,
                            slice_sizes=(1,),
                            mode=lax.GatherScatterMode.PROMISE_IN_BOUNDS)
                        for jj in range(128 // L):
                            sl = pl.ds(jj * L, L)
                            rows[b][bb + e, sl] = graw[b][bb + e, sl] * wvec

            loc = j * K - pbase
            for i in range(K // L):
                didx[b][pl.ds(i * L, L)] = didx_st[pl.ds(loc + i * L, L)]
            pltpu.async_copy(rows[b], acc_sh.at[didx[b]], ssem[b], add=True)
            if issue_src2:
                issue_src(j + 2, b)

        issue_src(jnp.int32(0), 0)
        issue_src(jnp.int32(1), 1)
        wait_src(jnp.int32(0), 0)
        issue_gather(jnp.int32(0), 0)

        chunk(jnp.int32(0), 0, 0, first=True)
        chunk(jnp.int32(1), 1, 0, first=True)

        @pl.loop(2, ncha, step=2)
        def _main_a(j):
            chunk(j, 0, 0)
            chunk(j + 1, 1, 0)

        pltpu.sync_copy(dst_hbm.at[pl.ds(ebase + ncha * K, nchb * K)],
                        didx_st.at[pl.ds(0, nchb * K)])

        @pl.loop(ncha, nch - 3, step=2)
        def _main_b(j):
            chunk(j, 0, ncha * K)
            chunk(j + 1, 1, ncha * K)

        chunk(jnp.int32(nch - 3), 0, ncha * K)
        chunk(jnp.int32(nch - 2), 1, ncha * K, issue_src2=False)
        chunk(jnp.int32(nch - 1), 0, ncha * K, issue_src2=False,
              issue_next=False)
        wait_scatter(1)
        wait_scatter(0)
        plsc.subcore_barrier()

        col = pl.multiple_of(c * 128, 128)

        @pl.loop(s, num_groups, step=NS)
        def _copy_out(g):
            pltpu.sync_copy(acc_sh.at[pl.ds(g * K, K)],
                            out_hbm.at[pl.ds(g * K, K), pl.ds(col, 128)])

    return seg_sum(x2, src, dst, w)


def _tc_combine(agg, x_0, W1):
    N, C = x_0.shape
    TN = 400
    assert N % TN == 0

    def body(a_ref, x0_ref, w_ref, out_ref):
        h = a_ref[...] * (1.0 - ALPHA) + ALPHA * x0_ref[...]
        out_ref[...] = (1.0 - BETA) * h + BETA * jnp.dot(
            h, w_ref[...], preferred_element_type=jnp.float32)

    return pl.pallas_call(
        body,
        grid=(N // TN,),
        in_specs=[
            pl.BlockSpec((TN, C), lambda i: (i, 0)),
            pl.BlockSpec((TN, C), lambda i: (i, 0)),
            pl.BlockSpec((C, C), lambda i: (0, 0)),
        ],
        out_specs=pl.BlockSpec((TN, C), lambda i: (i, 0)),
        out_shape=jax.ShapeDtypeStruct((N, C), jnp.float32),
    )(agg, x_0, W1)


def kernel(x, x_0, edge_index, edge_weight, W1, node_lock):
    N, C = x.shape
    assert C == 256
    x2 = x.reshape(2 * N, 128)
    src = edge_index[0]
    dst = edge_index[1]
    agg = _sc_segment_sum(x2, src, dst, edge_weight, N)
    return _tc_combine(agg, x_0, W1)


# speedup vs baseline: 2.4681x; 2.4681x over previous
import functools
from math import log

import jax
import jax.numpy as jnp
from jax import lax
from jax.experimental import pallas as pl
from jax.experimental.pallas import tpu as pltpu
from jax.experimental.pallas import tpu_sc as plsc

ALPHA = 0.1
BETA = log(0.5 / (2 + 1) + 1)

NC = 2
NS = 16
L = 16
NB = 3

K = 80


_DNUMS = lax.GatherDimensionNumbers(
    offset_dims=(), collapsed_slice_dims=(0,), start_index_map=(0,))


def _sc_segment_sum(x2, src, dst, w, N):
    E = src.shape[0]
    ept = E // NS
    nch = ept // K
    assert E % NS == 0 and ept % K == 0 and K % 8 == 0
    assert (nch - 2) % NB == 0 and nch >= 5
    num_groups = N // K
    assert N % K == 0
    mesh = plsc.VectorSubcoreMesh(core_axis_name="c", subcore_axis_name="s")

    @functools.partial(
        pl.kernel,
        mesh=mesh,
        out_type=jax.ShapeDtypeStruct((N, NC * 128), jnp.float32),
        scratch_types=[
            pltpu.VMEM((ept,), jnp.int32),
            pltpu.VMEM((NB, K), jnp.int32),
            pltpu.VMEM((NB, K), jnp.float32),
            pltpu.VMEM((K,), jnp.int32),
            pltpu.VMEM((K,), jnp.int32),
            pltpu.VMEM((K,), jnp.int32),
            pltpu.VMEM((K,), jnp.int32),
            pltpu.VMEM((K,), jnp.int32),
            pltpu.VMEM((K,), jnp.int32),
            pltpu.VMEM((K, 128), jnp.float32),
            pltpu.VMEM((K, 128), jnp.float32),
            pltpu.VMEM((K, 128), jnp.float32),
            pltpu.VMEM_SHARED((N, 128), jnp.float32),
            pltpu.SemaphoreType.DMA,
            pltpu.SemaphoreType.DMA,
            pltpu.SemaphoreType.DMA,
            pltpu.SemaphoreType.DMA,
            pltpu.SemaphoreType.DMA,
            pltpu.SemaphoreType.DMA,
            pltpu.SemaphoreType.DMA,
            pltpu.SemaphoreType.DMA,
            pltpu.SemaphoreType.DMA,
        ],
    )
    def seg_sum(x2_hbm, src_hbm, dst_hbm, w_hbm, out_hbm,
                didx_all, sidx, wch, gidx0, gidx1, gidx2,
                didx0, didx1, didx2, rows0, rows1, rows2, acc_sh,
                gsem0, gsem1, gsem2, csem0, csem1, csem2,
                ssem0, ssem1, ssem2):
        c = lax.axis_index("c")
        s = lax.axis_index("s")
        gidx = (gidx0, gidx1, gidx2)
        didx = (didx0, didx1, didx2)
        rows = (rows0, rows1, rows2)
        gsem = (gsem0, gsem1, gsem2)
        csem = (csem0, csem1, csem2)
        ssem = (ssem0, ssem1, ssem2)
        ebase = s * ept

        pltpu.sync_copy(dst_hbm.at[pl.ds(ebase, ept)], didx_all)

        zero16 = jnp.zeros((L,), jnp.float32)

        @pl.loop(0, K)
        def _zero_rows(r):
            for j in range(128 // L):
                rows0[r, pl.ds(j * L, L)] = zero16
                rows2[r, pl.ds(j * L, L)] = zero16

        @pl.loop(s, num_groups, step=NS)
        def _zero_acc(g):
            pltpu.sync_copy(rows0, acc_sh.at[pl.ds(g * K, K)])

        plsc.subcore_barrier()

        def issue_src(j, t):
            pltpu.async_copy(src_hbm.at[pl.ds(ebase + j * K, K)],
                             sidx.at[t], csem[t])
            pltpu.async_copy(w_hbm.at[pl.ds(ebase + j * K, K)],
                             wch.at[t], csem[t])

        def wait_src(j, t):
            pltpu.make_async_copy(src_hbm.at[pl.ds(ebase + j * K, K)],
                                  sidx.at[t], csem[t]).wait()
            pltpu.make_async_copy(w_hbm.at[pl.ds(ebase + j * K, K)],
                                  wch.at[t], csem[t]).wait()

        def issue_gather(j, t):
            for i in range(K // L):
                sl = pl.ds(i * L, L)
                gidx[t][sl] = sidx[t, sl] * 2 + c
            pltpu.async_copy(x2_hbm.at[gidx[t]], rows[t], gsem[t])

        def wait_gather(t):
            pltpu.make_async_copy(x2_hbm.at[gidx[t]], rows[t], gsem[t]).wait()

        def wait_scatter(t):
            pltpu.make_async_copy(rows[t], acc_sh.at[didx[t]], ssem[t]).wait()

        def chunk(j, t, issue_next=True, issue_src2=True):
            t1 = (t + 1) % NB
            if issue_next:
                wait_scatter(t1)
                wait_src(j + 1, t1)
                issue_gather(j + 1, t1)
            wait_gather(t)

            @pl.loop(0, K, step=L)
            def _wblk(bb):
                wv = wch[t, pl.ds(bb, L)]
                for e in range(L):
                    wvec = lax.gather(
                        wv, jnp.full((L, 1), e, jnp.int32), _DNUMS,
                        slice_sizes=(1,),
                        mode=lax.GatherScatterMode.PROMISE_IN_BOUNDS)
                    for jj in range(128 // L):
                        sl = pl.ds(jj * L, L)
                        rows[t][bb + e, sl] = rows[t][bb + e, sl] * wvec

            off = j * K
            for i in range(K // L):
                didx[t][pl.ds(i * L, L)] = didx_all[pl.ds(off + i * L, L)]
            pltpu.async_copy(rows[t], acc_sh.at[didx[t]], ssem[t], add=True)
            if issue_src2:
                issue_src(j + 2, (t + 2) % NB)

        izero = jnp.zeros((L,), jnp.int32)
        for i in range(K // L):
            didx0[pl.ds(i * L, L)] = izero
        pltpu.async_copy(rows2, acc_sh.at[didx0], ssem1, add=True)
        pltpu.async_copy(rows2, acc_sh.at[didx0], ssem2, add=True)

        issue_src(jnp.int32(0), 0)
        issue_src(jnp.int32(1), 1)
        wait_src(jnp.int32(0), 0)
        issue_gather(jnp.int32(0), 0)

        @pl.loop(0, nch - 2, step=NB)
        def _main(j):
            chunk(j, 0)
            chunk(j + 1, 1)
            chunk(j + 2, 2)

        chunk(jnp.int32(nch - 2), 0, issue_src2=False)
        chunk(jnp.int32(nch - 1), 1, issue_next=False, issue_src2=False)
        wait_scatter(2)
        wait_scatter(0)
        wait_scatter(1)
        plsc.subcore_barrier()

        col = pl.multiple_of(c * 128, 128)

        @pl.loop(s, num_groups, step=NS)
        def _copy_out(g):
            pltpu.sync_copy(acc_sh.at[pl.ds(g * K, K)],
                            out_hbm.at[pl.ds(g * K, K), pl.ds(col, 128)])

    return seg_sum(x2, src, dst, w)


def _tc_combine(agg, x_0, W1):
    N, C = x_0.shape
    TN = 400
    assert N % TN == 0

    def body(a_ref, x0_ref, w_ref, out_ref):
        h = a_ref[...] * (1.0 - ALPHA) + ALPHA * x0_ref[...]
        out_ref[...] = (1.0 - BETA) * h + BETA * jnp.dot(
            h, w_ref[...], preferred_element_type=jnp.float32)

    return pl.pallas_call(
        body,
        grid=(N // TN,),
        in_specs=[
            pl.BlockSpec((TN, C), lambda i: (i, 0)),
            pl.BlockSpec((TN, C), lambda i: (i, 0)),
            pl.BlockSpec((C, C), lambda i: (0, 0)),
        ],
        out_specs=pl.BlockSpec((TN, C), lambda i: (i, 0)),
        out_shape=jax.ShapeDtypeStruct((N, C), jnp.float32),
    )(agg, x_0, W1)


def kernel(x, x_0, edge_index, edge_weight, W1, node_lock):
    N, C = x.shape
    assert C == 256
    x2 = x.reshape(2 * N, 128)
    src = edge_index[0]
    dst = edge_index[1]
    agg = _sc_segment_sum(x2, src, dst, edge_weight, N)
    return _tc_combine(agg, x_0, W1)


# speedup vs baseline: 2.4714x; 1.0014x over previous
import functools
from math import log

import jax
import jax.numpy as jnp
from jax import lax
from jax.experimental import pallas as pl
from jax.experimental.pallas import tpu as pltpu
from jax.experimental.pallas import tpu_sc as plsc

ALPHA = 0.1
BETA = log(0.5 / (2 + 1) + 1)

NC = 2
NS = 16
L = 16
NB = 3

K = 80


_DNUMS = lax.GatherDimensionNumbers(
    offset_dims=(), collapsed_slice_dims=(0,), start_index_map=(0,))


def _sc_segment_sum(x2, src, dst, w, N):
    E = src.shape[0]
    ept = E // NS
    nch = ept // K
    assert E % NS == 0 and ept % K == 0 and K % 8 == 0
    assert (nch - 2) % NB == 0 and nch >= 5
    num_groups = N // K
    assert N % K == 0
    mesh = plsc.VectorSubcoreMesh(core_axis_name="c", subcore_axis_name="s")

    @functools.partial(
        pl.kernel,
        mesh=mesh,
        out_type=jax.ShapeDtypeStruct((N, NC * 128), jnp.float32),
        scratch_types=[
            pltpu.VMEM((ept // K, K), jnp.int32),
            pltpu.VMEM((NB, K), jnp.int32),
            pltpu.VMEM((NB, K), jnp.float32),
            pltpu.VMEM((K,), jnp.int32),
            pltpu.VMEM((K,), jnp.int32),
            pltpu.VMEM((K,), jnp.int32),
            pltpu.VMEM((K, 128), jnp.float32),
            pltpu.VMEM((K, 128), jnp.float32),
            pltpu.VMEM((K, 128), jnp.float32),
            pltpu.VMEM_SHARED((N, 128), jnp.float32),
            pltpu.SemaphoreType.DMA,
            pltpu.SemaphoreType.DMA,
            pltpu.SemaphoreType.DMA,
            pltpu.SemaphoreType.DMA,
            pltpu.SemaphoreType.DMA,
            pltpu.SemaphoreType.DMA,
            pltpu.SemaphoreType.DMA,
            pltpu.SemaphoreType.DMA,
            pltpu.SemaphoreType.DMA,
        ],
    )
    def seg_sum(x2_hbm, src_hbm, dst_hbm, w_hbm, out_hbm,
                didx_all, sidx, wch, gidx0, gidx1, gidx2,
                rows0, rows1, rows2, acc_sh,
                gsem0, gsem1, gsem2, csem0, csem1, csem2,
                ssem0, ssem1, ssem2):
        c = lax.axis_index("c")
        s = lax.axis_index("s")
        gidx = (gidx0, gidx1, gidx2)
        rows = (rows0, rows1, rows2)
        gsem = (gsem0, gsem1, gsem2)
        csem = (csem0, csem1, csem2)
        ssem = (ssem0, ssem1, ssem2)
        ebase = s * ept

        pltpu.sync_copy(dst_hbm.at[s], didx_all)

        zero16 = jnp.zeros((L,), jnp.float32)

        @pl.loop(0, K)
        def _zero_rows(r):
            for j in range(128 // L):
                rows0[r, pl.ds(j * L, L)] = zero16
                rows2[r, pl.ds(j * L, L)] = zero16

        @pl.loop(s, num_groups, step=NS)
        def _zero_acc(g):
            pltpu.sync_copy(rows0, acc_sh.at[pl.ds(g * K, K)])

        plsc.subcore_barrier()

        def issue_src(j, t):
            pltpu.async_copy(src_hbm.at[pl.ds(ebase + j * K, K)],
                             sidx.at[t], csem[t])
            pltpu.async_copy(w_hbm.at[pl.ds(ebase + j * K, K)],
                             wch.at[t], csem[t])

        def wait_src(j, t):
            pltpu.make_async_copy(src_hbm.at[pl.ds(ebase + j * K, K)],
                                  sidx.at[t], csem[t]).wait()
            pltpu.make_async_copy(w_hbm.at[pl.ds(ebase + j * K, K)],
                                  wch.at[t], csem[t]).wait()

        def issue_gather(j, t):
            for i in range(K // L):
                sl = pl.ds(i * L, L)
                gidx[t][sl] = sidx[t, sl] * 2 + c
            pltpu.async_copy(x2_hbm.at[gidx[t]], rows[t], gsem[t])

        def wait_gather(t):
            pltpu.make_async_copy(x2_hbm.at[gidx[t]], rows[t], gsem[t]).wait()

        def wait_scatter(t, j):
            pltpu.make_async_copy(rows[t], acc_sh.at[didx_all.at[j]],
                                  ssem[t]).wait()

        def chunk(j, t, issue_next=True, issue_src2=True):
            t1 = (t + 1) % NB
            if issue_next:
                wait_scatter(t1, jnp.maximum(j - 2, 0))
                wait_src(j + 1, t1)
                issue_gather(j + 1, t1)
            wait_gather(t)

            @pl.loop(0, K, step=L)
            def _wblk(bb):
                wv = wch[t, pl.ds(bb, L)]
                for e in range(L):
                    wvec = lax.gather(
                        wv, jnp.full((L, 1), e, jnp.int32), _DNUMS,
                        slice_sizes=(1,),
                        mode=lax.GatherScatterMode.PROMISE_IN_BOUNDS)
                    for jj in range(128 // L):
                        sl = pl.ds(jj * L, L)
                        rows[t][bb + e, sl] = rows[t][bb + e, sl] * wvec

            pltpu.async_copy(rows[t], acc_sh.at[didx_all.at[j]], ssem[t],
                             add=True)
            if issue_src2:
                issue_src(j + 2, (t + 2) % NB)

        pltpu.async_copy(rows2, acc_sh.at[didx_all.at[0]], ssem1, add=True)
        pltpu.async_copy(rows2, acc_sh.at[didx_all.at[0]], ssem2, add=True)

        issue_src(jnp.int32(0), 0)
        issue_src(jnp.int32(1), 1)
        wait_src(jnp.int32(0), 0)
        issue_gather(jnp.int32(0), 0)

        @pl.loop(0, nch - 2, step=NB)
        def _main(j):
            chunk(j, 0)
            chunk(j + 1, 1)
            chunk(j + 2, 2)

        chunk(jnp.int32(nch - 2), 0, issue_src2=False)
        chunk(jnp.int32(nch - 1), 1, issue_next=False, issue_src2=False)
        wait_scatter(2, jnp.int32(nch - 3))
        wait_scatter(0, jnp.int32(nch - 2))
        wait_scatter(1, jnp.int32(nch - 1))
        plsc.subcore_barrier()

        col = pl.multiple_of(c * 128, 128)

        @pl.loop(s, num_groups, step=NS)
        def _copy_out(g):
            pltpu.sync_copy(acc_sh.at[pl.ds(g * K, K)],
                            out_hbm.at[pl.ds(g * K, K), pl.ds(col, 128)])

    return seg_sum(x2, src, dst, w)


def _tc_combine(agg, x_0, W1):
    N, C = x_0.shape
    TN = 400
    assert N % TN == 0

    def body(a_ref, x0_ref, w_ref, out_ref):
        h = a_ref[...] * (1.0 - ALPHA) + ALPHA * x0_ref[...]
        out_ref[...] = (1.0 - BETA) * h + BETA * jnp.dot(
            h, w_ref[...], preferred_element_type=jnp.float32)

    return pl.pallas_call(
        body,
        grid=(N // TN,),
        in_specs=[
            pl.BlockSpec((TN, C), lambda i: (i, 0)),
            pl.BlockSpec((TN, C), lambda i: (i, 0)),
            pl.BlockSpec((C, C), lambda i: (0, 0)),
        ],
        out_specs=pl.BlockSpec((TN, C), lambda i: (i, 0)),
        out_shape=jax.ShapeDtypeStruct((N, C), jnp.float32),
    )(agg, x_0, W1)


def kernel(x, x_0, edge_index, edge_weight, W1, node_lock):
    N, C = x.shape
    assert C == 256
    x2 = x.reshape(2 * N, 128)
    src = edge_index[0]
    E = src.shape[0]
    dst3 = edge_index[1].reshape(NS, (E // NS) // K, K)
    agg = _sc_segment_sum(x2, src, dst3, edge_weight, N)
    return _tc_combine(agg, x_0, W1)


# speedup vs baseline: 2.5343x; 1.0254x over previous
import functools
from math import log

import jax
import jax.numpy as jnp
from jax import lax
from jax.experimental import pallas as pl
from jax.experimental.pallas import tpu as pltpu
from jax.experimental.pallas import tpu_sc as plsc

ALPHA = 0.1
BETA = log(0.5 / (2 + 1) + 1)

NC = 2
NS = 16
L = 16

K = 80


_DNUMS = lax.GatherDimensionNumbers(
    offset_dims=(), collapsed_slice_dims=(0,), start_index_map=(0,))


def _sc_segment_sum(x2, src, dst, w, N):
    E = src.shape[0]
    ept = E // NS
    nch = ept // K
    assert E % NS == 0 and ept % K == 0 and K % 8 == 0 and nch % 2 == 1
    num_groups = N // K
    assert N % K == 0
    mesh = plsc.VectorSubcoreMesh(core_axis_name="c", subcore_axis_name="s")

    @functools.partial(
        pl.kernel,
        mesh=mesh,
        out_type=jax.ShapeDtypeStruct((N, NC * 128), jnp.float32),
        scratch_types=[
            pltpu.VMEM((ept,), jnp.int32),
            pltpu.VMEM((ept,), jnp.float32),
            pltpu.VMEM((K,), jnp.int32),
            pltpu.VMEM((K,), jnp.int32),
            pltpu.VMEM((K,), jnp.int32),
            pltpu.VMEM((K,), jnp.int32),
            pltpu.VMEM((K,), jnp.int32),
            pltpu.VMEM((K,), jnp.int32),
            pltpu.VMEM((K, 128), jnp.float32),
            pltpu.VMEM((K, 128), jnp.float32),
            pltpu.VMEM_SHARED((N, 128), jnp.float32),
            pltpu.SemaphoreType.DMA,
            pltpu.SemaphoreType.DMA,
            pltpu.SemaphoreType.DMA,
            pltpu.SemaphoreType.DMA,
            pltpu.SemaphoreType.DMA,
            pltpu.SemaphoreType.DMA,
        ],
    )
    def seg_sum(x2_hbm, src_hbm, dst_hbm, w_hbm, out_hbm,
                didx_all, w_all, sidx0, sidx1, gidx0, gidx1, didx0, didx1,
                rows0, rows1, acc_sh, gsem0, gsem1, csem0, csem1,
                ssem0, ssem1):
        c = lax.axis_index("c")
        s = lax.axis_index("s")
        sidx = (sidx0, sidx1)
        gidx = (gidx0, gidx1)
        didx = (didx0, didx1)
        rows = (rows0, rows1)
        gsem = (gsem0, gsem1)
        csem = (csem0, csem1)
        ssem = (ssem0, ssem1)
        ebase = s * ept

        pltpu.sync_copy(dst_hbm.at[pl.ds(ebase, ept)], didx_all)
        pltpu.sync_copy(w_hbm.at[pl.ds(ebase, ept)], w_all)

        zero16 = jnp.zeros((L,), jnp.float32)

        @pl.loop(0, K)
        def _zero_rows(r):
            for j in range(128 // L):
                rows0[r, pl.ds(j * L, L)] = zero16

        @pl.loop(s, num_groups, step=NS)
        def _zero_acc(g):
            pltpu.sync_copy(rows0, acc_sh.at[pl.ds(g * K, K)])

        plsc.subcore_barrier()

        def issue_src(j, b):
            pltpu.async_copy(src_hbm.at[pl.ds(ebase + j * K, K)],
                             sidx[b], csem[b])

        def wait_src(j, b):
            pltpu.make_async_copy(src_hbm.at[pl.ds(ebase + j * K, K)],
                                  sidx[b], csem[b]).wait()

        def issue_gather(j, b):
            for i in range(K // L):
                sl = pl.ds(i * L, L)
                gidx[b][sl] = sidx[b][sl] * 2 + c
            pltpu.async_copy(x2_hbm.at[gidx[b]], rows[b], gsem[b])

        def wait_scatter(b):
            pltpu.make_async_copy(rows[b], acc_sh.at[didx[b]], ssem[b]).wait()

        def chunk(j, b, first=False, issue_src2=True, issue_next=True):
            off = j * K
            b2 = 1 - b
            if issue_src2:
                issue_src(j + 2, b)
            if not first:
                wait_scatter(b2)
            if issue_next:
                wait_src(j + 1, b2)
                issue_gather(j + 1, b2)
            pltpu.make_async_copy(x2_hbm.at[gidx[b]], rows[b], gsem[b]).wait()

            @pl.loop(0, K, step=L)
            def _wblk(bb):
                wv = w_all[pl.ds(off + bb, L)]
                for e in range(L):
                    wvec = lax.gather(
                        wv, jnp.full((L, 1), e, jnp.int32), _DNUMS,
                        slice_sizes=(1,),
                        mode=lax.GatherScatterMode.PROMISE_IN_BOUNDS)
                    for jj in range(128 // L):
                        sl = pl.ds(jj * L, L)
                        rows[b][bb + e, sl] = rows[b][bb + e, sl] * wvec

            for i in range(K // L):
                sl = pl.ds(i * L, L)
                didx[b][sl] = didx_all[pl.ds(off + i * L, L)]
            pltpu.async_copy(rows[b], acc_sh.at[didx[b]], ssem[b], add=True)

        issue_src(jnp.int32(0), 0)
        wait_src(jnp.int32(0), 0)
        issue_gather(jnp.int32(0), 0)
        issue_src(jnp.int32(1), 1)

        chunk(jnp.int32(0), 0, first=True)

        @pl.loop(1, nch - 2, step=2)
        def _main(j):
            chunk(j, 1)
            chunk(j + 1, 0)

        chunk(jnp.int32(nch - 2), 1, issue_src2=False)
        chunk(jnp.int32(nch - 1), 0, issue_src2=False, issue_next=False)
        wait_scatter(0)
        plsc.subcore_barrier()

        col = pl.multiple_of(c * 128, 128)

        @pl.loop(s, num_groups, step=NS)
        def _copy_out(g):
            pltpu.sync_copy(acc_sh.at[pl.ds(g * K, K)],
                            out_hbm.at[pl.ds(g * K, K), pl.ds(col, 128)])

    return seg_sum(x2, src, dst, w)


def _tc_combine(agg, x_0, W1):
    N, C = x_0.shape
    TN = 400
    assert N % TN == 0

    def body(a_ref, x0_ref, w_ref, out_ref):
        h = a_ref[...] * (1.0 - ALPHA) + ALPHA * x0_ref[...]
        out_ref[...] = (1.0 - BETA) * h + BETA * jnp.dot(
            h, w_ref[...], preferred_element_type=jnp.float32)

    return pl.pallas_call(
        body,
        grid=(N // TN,),
        in_specs=[
            pl.BlockSpec((TN, C), lambda i: (i, 0)),
            pl.BlockSpec((TN, C), lambda i: (i, 0)),
            pl.BlockSpec((C, C), lambda i: (0, 0)),
        ],
        out_specs=pl.BlockSpec((TN, C), lambda i: (i, 0)),
        out_shape=jax.ShapeDtypeStruct((N, C), jnp.float32),
    )(agg, x_0, W1)


def kernel(x, x_0, edge_index, edge_weight, W1, node_lock):
    N, C = x.shape
    assert C == 256
    x2 = x.reshape(2 * N, 128)
    src = edge_index[0]
    dst = edge_index[1]
    agg = _sc_segment_sum(x2, src, dst, edge_weight, N)
    return _tc_combine(agg, x_0, W1)
